# baseline (speedup 1.0000x reference)
import jax
import jax.numpy as jnp
from jax.experimental import pallas as pl
from jax.experimental.pallas import tpu as pltpu

_B = 16384
_D = 256
_NX = 8
_BLK = 2048


def _body(idx_ref, x_hbm, pos_hbm, out_hbm, row_v, blk_v, sem_x, sem_row,
          sem_pos):
    rows = _B // _NX
    xcp = [
        pltpu.make_async_copy(
            x_hbm.at[pl.ds(i * rows, rows), :],
            out_hbm.at[pl.ds(i * rows, rows), pl.ds(0, _D)],
            sem_x,
        )
        for i in range(_NX)
    ]
    for c in xcp:
        c.start()
    rcp = pltpu.make_async_copy(
        pos_hbm.at[pl.ds(idx_ref[0], 1), :], row_v, sem_row)
    rcp.start()
    rcp.wait()
    blk_v[...] = jnp.broadcast_to(row_v[...], blk_v.shape)
    pcp = [
        pltpu.make_async_copy(
            blk_v,
            out_hbm.at[pl.ds(i * _BLK, _BLK), pl.ds(_D, _D)],
            sem_pos,
        )
        for i in range(_B // _BLK)
    ]
    for c in pcp:
        c.start()
    for c in xcp:
        c.wait()
    for c in pcp:
        c.wait()


def kernel(x, pos_table, index):
    B, _, D = x.shape
    x2 = x.reshape(B, D)
    pos2 = pos_table.reshape(pos_table.shape[1], D)
    idx = jnp.asarray(index, jnp.int32).reshape(1)
    out = pl.pallas_call(
        _body,
        grid_spec=pltpu.PrefetchScalarGridSpec(
            num_scalar_prefetch=1,
            grid=(1,),
            in_specs=[
                pl.BlockSpec(memory_space=pltpu.HBM),
                pl.BlockSpec(memory_space=pltpu.HBM),
            ],
            out_specs=pl.BlockSpec(memory_space=pltpu.HBM),
            scratch_shapes=[
                pltpu.VMEM((1, D), jnp.float32),
                pltpu.VMEM((_BLK, D), jnp.float32),
                pltpu.SemaphoreType.DMA,
                pltpu.SemaphoreType.DMA,
                pltpu.SemaphoreType.DMA,
            ],
        ),
        out_shape=jax.ShapeDtypeStruct((B, 2 * D), jnp.float32),
    )(idx, x2, pos2)
    return out.reshape(B, 1, 2 * D)


# speedup vs baseline: 6.3161x; 6.3161x over previous
import jax
import jax.numpy as jnp
from jax.experimental import pallas as pl
from jax.experimental.pallas import tpu as pltpu

_BM = 512
_D = 256


def _body(idx_ref, x_ref, pos_ref, out_ref, pos_full):
    del idx_ref

    @pl.when(pl.program_id(0) == 0)
    def _():
        pos_full[...] = jnp.broadcast_to(pos_ref[0], (_BM, _D))

    out_ref[:, 0:_D] = x_ref[...]
    out_ref[:, _D:2 * _D] = pos_full[...]


def kernel(x, pos_table, index):
    B, _, D = x.shape
    x2 = x.reshape(B, D)
    pos3 = pos_table.reshape(pos_table.shape[1], 1, D)
    grid = B // _BM
    idx = jnp.asarray(index, jnp.int32).reshape(1)
    out = pl.pallas_call(
        _body,
        grid_spec=pltpu.PrefetchScalarGridSpec(
            num_scalar_prefetch=1,
            grid=(grid,),
            in_specs=[
                pl.BlockSpec((_BM, D), lambda i, s: (i, 0)),
                pl.BlockSpec((1, 1, D), lambda i, s: (s[0], 0, 0)),
            ],
            out_specs=pl.BlockSpec((_BM, 2 * D), lambda i, s: (i, 0)),
            scratch_shapes=[
                pltpu.VMEM((_BM, D), jnp.float32),
            ],
        ),
        out_shape=jax.ShapeDtypeStruct((B, 2 * D), jnp.float32),
        compiler_params=pltpu.CompilerParams(
            dimension_semantics=("arbitrary",),
        ),
    )(idx, x2, pos3)
    return out.reshape(B, 1, 2 * D)


# speedup vs baseline: 17.3785x; 2.7515x over previous
import jax
import jax.numpy as jnp
from jax.experimental import pallas as pl
from jax.experimental.pallas import tpu as pltpu

_BM = 512
_D = 256


def _body(idx_ref, x_ref, pos_ref, out_ref, pos_full):
    @pl.when(pl.program_id(0) == 0)
    def _():
        row = idx_ref[0] % 8
        pos_full[...] = jnp.broadcast_to(
            pos_ref[0, pl.ds(row, 1), :], (_BM, _D))

    out_ref[:, 0, 0:_D] = x_ref[:, 0, :]
    out_ref[:, 0, _D:2 * _D] = pos_full[...]


def kernel(x, pos_table, index):
    B, _, D = x.shape
    grid = B // _BM
    idx = jnp.asarray(index, jnp.int32).reshape(1)
    return pl.pallas_call(
        _body,
        grid_spec=pltpu.PrefetchScalarGridSpec(
            num_scalar_prefetch=1,
            grid=(grid,),
            in_specs=[
                pl.BlockSpec((_BM, 1, D), lambda i, s: (i, 0, 0)),
                pl.BlockSpec((1, 8, D), lambda i, s: (0, s[0] // 8, 0)),
            ],
            out_specs=pl.BlockSpec((_BM, 1, 2 * D), lambda i, s: (i, 0, 0)),
            scratch_shapes=[
                pltpu.VMEM((_BM, D), jnp.float32),
            ],
        ),
        out_shape=jax.ShapeDtypeStruct((B, 1, 2 * D), jnp.float32),
        compiler_params=pltpu.CompilerParams(
            dimension_semantics=("arbitrary",),
        ),
    )(idx, x, pos_table)


# speedup vs baseline: 22.5455x; 1.2973x over previous
import jax
import jax.numpy as jnp
from jax.experimental import pallas as pl
from jax.experimental.pallas import tpu as pltpu

_BM = 1024
_D = 256


def _body(idx_ref, x_ref, pos_ref, out_ref, pos_full):
    @pl.when(pl.program_id(0) == 0)
    def _():
        row = idx_ref[0] % 8
        pos_full[...] = jnp.broadcast_to(
            pos_ref[0, pl.ds(row, 1), :], (_BM, _D))

    out_ref[:, 0, 0:_D] = x_ref[:, 0, :]
    out_ref[:, 0, _D:2 * _D] = pos_full[...]


def kernel(x, pos_table, index):
    B, _, D = x.shape
    grid = B // _BM
    idx = jnp.asarray(index, jnp.int32).reshape(1)
    return pl.pallas_call(
        _body,
        grid_spec=pltpu.PrefetchScalarGridSpec(
            num_scalar_prefetch=1,
            grid=(grid,),
            in_specs=[
                pl.BlockSpec((_BM, 1, D), lambda i, s: (i, 0, 0)),
                pl.BlockSpec((1, 8, D), lambda i, s: (0, s[0] // 8, 0)),
            ],
            out_specs=pl.BlockSpec((_BM, 1, 2 * D), lambda i, s: (i, 0, 0)),
            scratch_shapes=[
                pltpu.VMEM((_BM, D), jnp.float32),
            ],
        ),
        out_shape=jax.ShapeDtypeStruct((B, 1, 2 * D), jnp.float32),
        compiler_params=pltpu.CompilerParams(
            dimension_semantics=("parallel",),
        ),
    )(idx, x, pos_table)


# speedup vs baseline: 26.9025x; 1.1933x over previous
import jax
import jax.numpy as jnp
from jax.experimental import pallas as pl
from jax.experimental.pallas import tpu as pltpu

_BM = 2048
_D = 256


def _body(idx_ref, x_ref, pos_ref, out_ref, pos_full):
    @pl.when(pl.program_id(0) == 0)
    def _():
        row = idx_ref[0] % 8
        pos_full[...] = jnp.broadcast_to(
            pos_ref[0, pl.ds(row, 1), :], (_BM, _D))

    out_ref[:, 0, 0:_D] = x_ref[:, 0, :]
    out_ref[:, 0, _D:2 * _D] = pos_full[...]


def kernel(x, pos_table, index):
    B, _, D = x.shape
    grid = B // _BM
    idx = jnp.asarray(index, jnp.int32).reshape(1)
    return pl.pallas_call(
        _body,
        grid_spec=pltpu.PrefetchScalarGridSpec(
            num_scalar_prefetch=1,
            grid=(grid,),
            in_specs=[
                pl.BlockSpec((_BM, 1, D), lambda i, s: (i, 0, 0)),
                pl.BlockSpec((1, 8, D), lambda i, s: (0, s[0] // 8, 0)),
            ],
            out_specs=pl.BlockSpec((_BM, 1, 2 * D), lambda i, s: (i, 0, 0)),
            scratch_shapes=[
                pltpu.VMEM((_BM, D), jnp.float32),
            ],
        ),
        out_shape=jax.ShapeDtypeStruct((B, 1, 2 * D), jnp.float32),
        compiler_params=pltpu.CompilerParams(
            dimension_semantics=("parallel",),
        ),
    )(idx, x, pos_table)


# speedup vs baseline: 27.5174x; 1.0229x over previous
import jax
import jax.numpy as jnp
from jax.experimental import pallas as pl
from jax.experimental.pallas import tpu as pltpu

_BM = 4096
_D = 256


def _body(idx_ref, x_ref, pos_ref, out_ref, pos_full):
    @pl.when(pl.program_id(0) == 0)
    def _():
        row = idx_ref[0] % 8
        pos_full[...] = jnp.broadcast_to(
            pos_ref[0, pl.ds(row, 1), :], (_BM, _D))

    out_ref[:, 0, 0:_D] = x_ref[:, 0, :]
    out_ref[:, 0, _D:2 * _D] = pos_full[...]


def kernel(x, pos_table, index):
    B, _, D = x.shape
    grid = B // _BM
    idx = jnp.asarray(index, jnp.int32).reshape(1)
    return pl.pallas_call(
        _body,
        grid_spec=pltpu.PrefetchScalarGridSpec(
            num_scalar_prefetch=1,
            grid=(grid,),
            in_specs=[
                pl.BlockSpec((_BM, 1, D), lambda i, s: (i, 0, 0)),
                pl.BlockSpec((1, 8, D), lambda i, s: (0, s[0] // 8, 0)),
            ],
            out_specs=pl.BlockSpec((_BM, 1, 2 * D), lambda i, s: (i, 0, 0)),
            scratch_shapes=[
                pltpu.VMEM((_BM, D), jnp.float32),
            ],
        ),
        out_shape=jax.ShapeDtypeStruct((B, 1, 2 * D), jnp.float32),
        compiler_params=pltpu.CompilerParams(
            dimension_semantics=("parallel",),
        ),
    )(idx, x, pos_table)
